# baseline (speedup 1.0000x reference)
import jax
import jax.numpy as jnp
from jax import lax
from jax.experimental import pallas as pl
from jax.experimental.pallas import tpu as pltpu
from jax.experimental.pallas import tpu_sc as plsc

_AH = 7
_AW = 7
_NPP = _AH * _AW
_SCALE = 0.125

_B, _C, _H, _W = 4, 256, 64, 64
_N = 1000
_HC = _C // 2
_NC, _NS, _L = 2, 16, 16
_NWORK = _NC * _NS
_RPW = 31
_EXTRA = _N - _RPW * _NWORK
_SLOTS = 64
_GLEN = 56
_HGROUPS = _HC // _L
_OUT_HALF = _HC * _NPP


def _sc_body(table, rois, out, rois_v,
             ix0, ix1, ix2, ix3, w0, w1, w2, w3,
             ul0, ur0, dl0, dr0, ul1, ur1, dl1, dr1,
             ov0, ov1, sem_g, sem_o):
    wid = lax.axis_index("s") * _NC + lax.axis_index("c")
    base_roi = wid * _RPW + jnp.minimum(wid, _EXTRA)
    n_rois = _RPW + jnp.where(wid < _EXTRA, 1, 0)

    pltpu.sync_copy(rois, rois_v)

    lanes = lax.iota(jnp.int32, _L)
    lanes49 = lanes * _NPP
    idx_refs = (ix0, ix1, ix2, ix3)
    w_refs = (w0, w1, w2, w3)
    bufs = ((ul0, ur0, dl0, dr0), (ul1, ur1, dl1, dr1))
    outs = (ov0, ov1)

    def meta_for(rt, pm):
        n5 = (base_roi + rt) * 5

        def mbody(j, cc):
            slot = j * _L
            within = slot + lanes
            pad_ok = within < _NPP
            ph = lax.div(within, _AW)
            pw = within - ph * _AW
            nv = jnp.full((_L,), n5, jnp.int32)
            bf = plsc.load_gather(rois_v, [nv])
            x1 = plsc.load_gather(rois_v, [nv + 1])
            y1 = plsc.load_gather(rois_v, [nv + 2])
            x2 = plsc.load_gather(rois_v, [nv + 3])
            y2 = plsc.load_gather(rois_v, [nv + 4])
            sw = x1 * _SCALE
            sh = y1 * _SCALE
            roi_w = jnp.maximum(x2 * _SCALE - sw, 0.0)
            roi_h = jnp.maximum(y2 * _SCALE - sh, 0.0)
            bin_w = roi_w / (_AW - 1.0)
            bin_h = roi_h / (_AH - 1.0)
            hh = sh + ph.astype(jnp.float32) * bin_h
            ww = sw + pw.astype(jnp.float32) * bin_w
            valid = (hh >= 0.0) & (hh < _H) & (ww >= 0.0) & (ww < _W) & pad_ok
            hi = jnp.clip(hh.astype(jnp.int32), 0, _H - 2)
            wi = jnp.clip(ww.astype(jnp.int32), 0, _W - 2)
            hr = hh - hi.astype(jnp.float32)
            wr = ww - wi.astype(jnp.float32)
            vf = jnp.where(valid, 1.0, 0.0)
            bi = bf.astype(jnp.int32)
            pix2 = (bi * (_H * _W) + hi * _W + wi) * 2
            for cref, coff in zip(idx_refs, (0, 2, 2 * _W, 2 * _W + 2)):
                cref[pl.ds(pm * 2 * _SLOTS + slot, _L)] = pix2 + coff
                cref[pl.ds(pm * 2 * _SLOTS + _SLOTS + slot, _L)] = pix2 + coff + 1
            w0[pl.ds(pm * _SLOTS + slot, _L)] = (1.0 - hr) * (1.0 - wr) * vf
            w1[pl.ds(pm * _SLOTS + slot, _L)] = (1.0 - hr) * wr * vf
            w2[pl.ds(pm * _SLOTS + slot, _L)] = hr * (1.0 - wr) * vf
            w3[pl.ds(pm * _SLOTS + slot, _L)] = hr * wr * vf
            return cc

        lax.fori_loop(0, _SLOTS // _L, mbody, 0)

    def issue_gathers(par, h, bset):
        off = par * 2 * _SLOTS + h * _SLOTS
        for cref, dst in zip(idx_refs, bset):
            pltpu.async_copy(table.at[cref.at[pl.ds(off, _GLEN)]], dst, sem_g)

    def half_body(r, h):
        bset = bufs[h]
        ov = outs[h]
        par = r & 1
        for dst in bset:
            pltpu.make_async_copy(table.at[pl.ds(0, _GLEN)], dst, sem_g).wait()

        @pl.when(r > 0)
        def _drain_out():
            pltpu.make_async_copy(ov, out.at[2 * base_roi + h], sem_o).wait()

        def do_point(p, cc):
            pv = jnp.full((_L,), par * _SLOTS + p, jnp.int32)
            a0 = plsc.load_gather(w0, [pv])
            a1 = plsc.load_gather(w1, [pv])
            a2 = plsc.load_gather(w2, [pv])
            a3 = plsc.load_gather(w3, [pv])
            pidx = lanes49 + p
            for g in range(_HGROUPS):
                sl = pl.ds(g * _L, _L)
                acc = (bset[0][p, sl] * a0 + bset[1][p, sl] * a1
                       + bset[2][p, sl] * a2 + bset[3][p, sl] * a3)
                plsc.store_scatter(ov, [pidx + g * (_L * _NPP)], acc)
            return cc

        lax.fori_loop(0, _NPP, do_point, 0)
        issue_gathers(1 - par, h, bset)
        pltpu.async_copy(ov, out.at[2 * (base_roi + r) + h], sem_o)

    def do_roi(r, carry):
        meta_for(jnp.minimum(r + 1, n_rois - 1), (r + 1) & 1)
        half_body(r, 0)
        half_body(r, 1)
        return carry

    meta_for(0, 0)
    issue_gathers(0, 0, bufs[0])
    issue_gathers(0, 1, bufs[1])
    lax.fori_loop(0, n_rois, do_roi, 0)

    for bset in bufs:
        for dst in bset:
            pltpu.make_async_copy(table.at[pl.ds(0, _GLEN)], dst, sem_g).wait()
    for h in (0, 1):
        pltpu.make_async_copy(outs[h], out.at[2 * base_roi + h], sem_o).wait()


def _build_sc_call():
    return pl.kernel(
        _sc_body,
        out_type=jax.ShapeDtypeStruct((2 * _N, _OUT_HALF), jnp.float32),
        mesh=plsc.VectorSubcoreMesh(core_axis_name="c", subcore_axis_name="s"),
        compiler_params=pltpu.CompilerParams(needs_layout_passes=False),
        scratch_types=[
            pltpu.VMEM((_N * 5,), jnp.float32),
            pltpu.VMEM((4 * _SLOTS,), jnp.int32),
            pltpu.VMEM((4 * _SLOTS,), jnp.int32),
            pltpu.VMEM((4 * _SLOTS,), jnp.int32),
            pltpu.VMEM((4 * _SLOTS,), jnp.int32),
            pltpu.VMEM((2 * _SLOTS,), jnp.float32),
            pltpu.VMEM((2 * _SLOTS,), jnp.float32),
            pltpu.VMEM((2 * _SLOTS,), jnp.float32),
            pltpu.VMEM((2 * _SLOTS,), jnp.float32),
            pltpu.VMEM((_GLEN, _HC), jnp.float32),
            pltpu.VMEM((_GLEN, _HC), jnp.float32),
            pltpu.VMEM((_GLEN, _HC), jnp.float32),
            pltpu.VMEM((_GLEN, _HC), jnp.float32),
            pltpu.VMEM((_GLEN, _HC), jnp.float32),
            pltpu.VMEM((_GLEN, _HC), jnp.float32),
            pltpu.VMEM((_GLEN, _HC), jnp.float32),
            pltpu.VMEM((_GLEN, _HC), jnp.float32),
            pltpu.VMEM((_OUT_HALF,), jnp.float32),
            pltpu.VMEM((_OUT_HALF,), jnp.float32),
            pltpu.SemaphoreType.DMA,
            pltpu.SemaphoreType.DMA,
        ],
    )


def kernel(features, rois):
    table = jnp.transpose(features, (0, 2, 3, 1)).reshape(_B * _H * _W * 2, _HC)
    flat = _build_sc_call()(table, rois.reshape(_N * 5))
    return flat.reshape(_N, _C, _AH, _AW)


# speedup vs baseline: 2.6572x; 2.6572x over previous
import jax
import jax.numpy as jnp
from jax import lax
from jax.experimental import pallas as pl
from jax.experimental.pallas import tpu as pltpu
from jax.experimental.pallas import tpu_sc as plsc

_AH = 7
_AW = 7
_SCALE = 0.125

_B, _C, _H, _W = 4, 256, 64, 64
_N = 1000
_PTS = _N * _AH * _AW
_NC, _NS, _L = 2, 16, 16
_NWORK = _NC * _NS
_PTS_PER_W = 1536
_PTS_PAD = _NWORK * _PTS_PER_W
_CHUNK = 32
_NCHUNKS = _PTS_PER_W // _CHUNK
_GROUPS = _C // _L


def _sc_body(table, rois, out, rois_v,
             idx0, idx1, idx2, idx3, w0, w1, w2, w3,
             ulA, urA, dlA, drA, ulB, urB, dlB, drB,
             ovA, ovB, sem_g, sem_o):
    wid = lax.axis_index("s") * _NC + lax.axis_index("c")
    base_pt = wid * _PTS_PER_W

    pltpu.sync_copy(rois, rois_v)

    lanes = lax.iota(jnp.int32, _L)
    idx_refs = (idx0, idx1, idx2, idx3)
    bufs = ((ulA, urA, dlA, drA), (ulB, urB, dlB, drB))
    outs = (ovA, ovB)

    def compute_meta(i, carry):
        p_local = i * _L
        p = jnp.full((_L,), base_pt, jnp.int32) + p_local + lanes
        n_raw = lax.div(p, 49)
        r = p - n_raw * 49
        ph = lax.div(r, 7)
        pw = r - ph * 7
        pad_ok = p < _PTS
        n5 = jnp.minimum(n_raw, _N - 1) * 5
        bf = plsc.load_gather(rois_v, [n5])
        x1 = plsc.load_gather(rois_v, [n5 + 1])
        y1 = plsc.load_gather(rois_v, [n5 + 2])
        x2 = plsc.load_gather(rois_v, [n5 + 3])
        y2 = plsc.load_gather(rois_v, [n5 + 4])
        sw = x1 * _SCALE
        sh = y1 * _SCALE
        roi_w = jnp.maximum(x2 * _SCALE - sw, 0.0)
        roi_h = jnp.maximum(y2 * _SCALE - sh, 0.0)
        bin_w = roi_w / (_AW - 1.0)
        bin_h = roi_h / (_AH - 1.0)
        hh = sh + ph.astype(jnp.float32) * bin_h
        ww = sw + pw.astype(jnp.float32) * bin_w
        valid = (hh >= 0.0) & (hh < _H) & (ww >= 0.0) & (ww < _W) & pad_ok
        hi = jnp.clip(hh.astype(jnp.int32), 0, _H - 2)
        wi = jnp.clip(ww.astype(jnp.int32), 0, _W - 2)
        hr = hh - hi.astype(jnp.float32)
        wr = ww - wi.astype(jnp.float32)
        vf = jnp.where(valid, 1.0, 0.0)
        bi = bf.astype(jnp.int32)
        base_idx = bi * (_H * _W) + hi * _W + wi
        sl = pl.ds(p_local, _L)
        idx0[sl] = base_idx
        idx1[sl] = base_idx + 1
        idx2[sl] = base_idx + _W
        idx3[sl] = base_idx + _W + 1
        w0[sl] = (1.0 - hr) * (1.0 - wr) * vf
        w1[sl] = (1.0 - hr) * wr * vf
        w2[sl] = hr * (1.0 - wr) * vf
        w3[sl] = hr * wr * vf
        return carry

    lax.fori_loop(0, _PTS_PER_W // _L, compute_meta, 0)

    def issue_gathers(c, bset):
        off = c * _CHUNK
        for cref, dst in zip(idx_refs, bset):
            pltpu.async_copy(table.at[cref.at[pl.ds(off, _CHUNK)]], dst, sem_g)

    def half_body(c, par):
        bset = bufs[par]
        ov = outs[par]
        for dst in bset:
            pltpu.make_async_copy(table.at[pl.ds(0, _CHUNK)], dst, sem_g).wait()
        pltpu.make_async_copy(ov, out.at[pl.ds(base_pt, _CHUNK)], sem_o).wait()

        def do_point(p, cc):
            pv = jnp.full((_L,), c * _CHUNK + p, jnp.int32)
            a0 = plsc.load_gather(w0, [pv])
            a1 = plsc.load_gather(w1, [pv])
            a2 = plsc.load_gather(w2, [pv])
            a3 = plsc.load_gather(w3, [pv])
            for g in range(_GROUPS):
                sl = pl.ds(g * _L, _L)
                ov[p, sl] = (bset[0][p, sl] * a0 + bset[1][p, sl] * a1
                             + bset[2][p, sl] * a2 + bset[3][p, sl] * a3)
            return cc

        lax.fori_loop(0, _CHUNK, do_point, 0)
        issue_gathers(jnp.minimum(c + 2, _NCHUNKS - 1), bset)
        pltpu.async_copy(ov, out.at[pl.ds(base_pt + c * _CHUNK, _CHUNK)], sem_o)

    issue_gathers(0, bufs[0])
    issue_gathers(1, bufs[1])
    pltpu.async_copy(ovA, out.at[pl.ds(base_pt, _CHUNK)], sem_o)
    pltpu.async_copy(ovB, out.at[pl.ds(base_pt + _CHUNK, _CHUNK)], sem_o)

    def do_pair(k, carry):
        half_body(k * 2, 0)
        half_body(k * 2 + 1, 1)
        return carry

    lax.fori_loop(0, _NCHUNKS // 2, do_pair, 0)

    for bset in bufs:
        for dst in bset:
            pltpu.make_async_copy(table.at[pl.ds(0, _CHUNK)], dst, sem_g).wait()
    for ov in outs:
        pltpu.make_async_copy(ov, out.at[pl.ds(base_pt, _CHUNK)], sem_o).wait()


def _build_sc_call():
    cbuf = pltpu.VMEM((_CHUNK, _C), jnp.float32)
    return pl.kernel(
        _sc_body,
        out_type=jax.ShapeDtypeStruct((_PTS_PAD, _C), jnp.float32),
        mesh=plsc.VectorSubcoreMesh(core_axis_name="c", subcore_axis_name="s"),
        compiler_params=pltpu.CompilerParams(needs_layout_passes=False),
        scratch_types=[
            pltpu.VMEM((_N * 5,), jnp.float32),
            pltpu.VMEM((_PTS_PER_W,), jnp.int32),
            pltpu.VMEM((_PTS_PER_W,), jnp.int32),
            pltpu.VMEM((_PTS_PER_W,), jnp.int32),
            pltpu.VMEM((_PTS_PER_W,), jnp.int32),
            pltpu.VMEM((_PTS_PER_W,), jnp.float32),
            pltpu.VMEM((_PTS_PER_W,), jnp.float32),
            pltpu.VMEM((_PTS_PER_W,), jnp.float32),
            pltpu.VMEM((_PTS_PER_W,), jnp.float32),
            cbuf, cbuf, cbuf, cbuf, cbuf, cbuf, cbuf, cbuf,
            cbuf, cbuf,
            pltpu.SemaphoreType.DMA,
            pltpu.SemaphoreType.DMA,
        ],
    )


def kernel(features, rois):
    table = jnp.transpose(features, (0, 2, 3, 1)).reshape(_B * _H * _W, _C)
    flat = _build_sc_call()(table, rois.reshape(_N * 5))
    out = flat[:_PTS].reshape(_N, _AH * _AW, _C)
    return jnp.transpose(out, (0, 2, 1)).reshape(_N, _C, _AH, _AW)
